# baseline (device time: 83385 ns/iter reference)
import jax
import jax.numpy as jnp
from jax import lax
from jax.experimental import pallas as pl
from jax.experimental.pallas import tpu as pltpu

N_DEV = 4
S = 1024
D = 2048
DC = 128
DC_ALL = N_DEV * DC
H = 16
HL = H // N_DEV
DH = 128
HB = HL * DH
DR = 32
SCALE = (DH + DR) ** -0.5
F32 = jnp.float32
BF16 = jnp.bfloat16


def _peer_barrier(peers):
    barrier_sem = pltpu.get_barrier_semaphore()
    for nbr in peers:
        pl.semaphore_signal(barrier_sem, inc=1, device_id=(nbr,),
                            device_id_type=pl.DeviceIdType.MESH)
    pl.semaphore_wait(barrier_sem, len(peers))


def _gather_body(x_ref, wdkv_ref, wuk_ref, wuv_ref, wqb_ref, wqrb_ref,
                 wkr_ref, wo_ref,
                 c_out, uk_out, uv_out, q_out, qr_out, kr_out, wob_out,
                 uk_b, uv_b, xb, wo_s, wo_stage,
                 send_sems, recv_sems, din_sem, dout_sem):
    my = lax.axis_index("i")
    barrier_sem = pltpu.get_barrier_semaphore()
    for j in range(1, N_DEV):
        pl.semaphore_signal(barrier_sem, inc=1,
                            device_id=((my + j) % N_DEV,),
                            device_id_type=pl.DeviceIdType.MESH)
    for i in range(N_DEV):
        din = pltpu.make_async_copy(
            wo_ref.at[pl.ds(i * HB, HB), :], wo_s, din_sem)
        din.start()
        din.wait()
        wo_stage[...] = wo_s[...].astype(BF16)
        dout = pltpu.make_async_copy(
            wo_stage, wob_out.at[pl.ds(i * HB, HB), :], dout_sem)
        dout.start()
        dout.wait()
    pl.semaphore_wait(barrier_sem, N_DEV - 1)

    uk_b[...] = wuk_ref[...].astype(BF16)
    uv_b[...] = wuv_ref[...].astype(BF16)
    sends = []
    for p_rel in range(1, N_DEV):
        p = (my + p_rel) % N_DEV
        for t, (w_b, dst) in enumerate(((uk_b, uk_out), (uv_b, uv_out))):
            r = pltpu.make_async_remote_copy(
                src_ref=w_b.at[:, pl.ds(p * HB, HB)],
                dst_ref=dst.at[pl.ds(my * DC, DC), :],
                send_sem=send_sems.at[p_rel - 1, t],
                recv_sem=recv_sems.at[3 - p_rel, t],
                device_id=(p,),
                device_id_type=pl.DeviceIdType.MESH,
            )
            r.start()
            sends.append(r)

    xb[...] = x_ref[...].astype(BF16)
    xbv = xb[...]
    c = jnp.dot(xbv, wdkv_ref[...].astype(BF16), preferred_element_type=F32)
    c_out[:, pl.ds(my * DC, DC)] = c.astype(BF16)
    for p_rel in range(1, N_DEV):
        p = (my + p_rel) % N_DEV
        r = pltpu.make_async_remote_copy(
            src_ref=c_out.at[:, pl.ds(my * DC, DC)],
            dst_ref=c_out.at[:, pl.ds(my * DC, DC)],
            send_sem=send_sems.at[p_rel - 1, 2],
            recv_sem=recv_sems.at[3 - p_rel, 2],
            device_id=(p,),
            device_id_type=pl.DeviceIdType.MESH,
        )
        r.start()
        sends.append(r)

    uk_out[pl.ds(my * DC, DC), :] = uk_b[:, pl.ds(my * HB, HB)]
    uv_out[pl.ds(my * DC, DC), :] = uv_b[:, pl.ds(my * HB, HB)]
    kr_out[...] = jnp.dot(xbv, wkr_ref[...].astype(BF16),
                          preferred_element_type=F32).astype(BF16)
    q_out[...] = jnp.dot(xbv, wqb_ref[...],
                         preferred_element_type=F32).astype(BF16)
    qr_out[...] = jnp.dot(xbv, wqrb_ref[...],
                          preferred_element_type=F32).astype(BF16)

    for r_slot in range(N_DEV - 1):
        o = (my + r_slot + 1) % N_DEV
        for t, dst in enumerate((uk_out, uv_out)):
            rcv = pltpu.make_async_remote_copy(
                src_ref=dst.at[pl.ds(o * DC, DC), :],
                dst_ref=dst.at[pl.ds(o * DC, DC), :],
                send_sem=send_sems.at[r_slot, t],
                recv_sem=recv_sems.at[r_slot, t],
                device_id=(my,),
                device_id_type=pl.DeviceIdType.MESH,
            )
            rcv.wait_recv()
        rcv = pltpu.make_async_remote_copy(
            src_ref=c_out.at[:, pl.ds(o * DC, DC)],
            dst_ref=c_out.at[:, pl.ds(o * DC, DC)],
            send_sem=send_sems.at[r_slot, 2],
            recv_sem=recv_sems.at[r_slot, 2],
            device_id=(my,),
            device_id_type=pl.DeviceIdType.MESH,
        )
        rcv.wait_recv()
    for s in sends:
        s.wait_send()


def _bc_body(c_ref, uk_ref, uv_ref, q_ref, qr_ref, kr_ref, wob_ref,
             out_ref, o_loc, o_recv, wo_v,
             send_sems, recv_sems, dma_sems):
    my = lax.axis_index("i")

    def wo_dma(i, buf):
        idx = (my + i) % N_DEV
        return pltpu.make_async_copy(
            wob_ref.at[pl.ds(idx * HB, HB), :],
            wo_v.at[buf],
            dma_sems.at[buf],
        )

    dma0 = wo_dma(0, 0)
    dma0.start()
    _peer_barrier([(my + j) % N_DEV for j in range(1, N_DEV)])

    cv = c_ref[...]
    krv = kr_ref[...]
    sends = []
    for h in range(HL):
        k_h = jnp.dot(cv, uk_ref[:, h * DH:(h + 1) * DH],
                      preferred_element_type=F32).astype(BF16)
        v_h = jnp.dot(cv, uv_ref[:, h * DH:(h + 1) * DH],
                      preferred_element_type=F32).astype(BF16)
        q_h = q_ref[:, h * DH:(h + 1) * DH]
        qr_h = qr_ref[:, h * DR:(h + 1) * DR]
        s = lax.dot_general(q_h, k_h, (((1,), (1,)), ((), ())),
                            preferred_element_type=F32)
        s += lax.dot_general(qr_h, krv, (((1,), (1,)), ((), ())),
                             preferred_element_type=F32)
        s *= SCALE
        m = jnp.max(s, axis=1, keepdims=True)
        p = jnp.exp(s - m)
        p = (p / jnp.sum(p, axis=1, keepdims=True)).astype(BF16)
        o_loc[:, h * DH:(h + 1) * DH] = jnp.dot(
            p, v_h, preferred_element_type=F32).astype(BF16)
        for p_rel in range(1, N_DEV):
            pd = (my + p_rel) % N_DEV
            r = pltpu.make_async_remote_copy(
                src_ref=o_loc.at[:, pl.ds(h * DH, DH)],
                dst_ref=o_recv.at[3 - p_rel, :, pl.ds(h * DH, DH)],
                send_sem=send_sems.at[p_rel - 1, h],
                recv_sem=recv_sems.at[3 - p_rel, h],
                device_id=(pd,),
                device_id_type=pl.DeviceIdType.MESH,
            )
            r.start()
            sends.append(r)

    dma0.wait()
    dma1 = wo_dma(1, 1)
    dma1.start()
    olv = o_loc[...]
    for cb in range(2):
        csl = pl.ds(cb * (D // 2), D // 2)
        out_ref[:, csl] = jnp.dot(olv, wo_v[0][:, cb * (D // 2):
                                               (cb + 1) * (D // 2)],
                                  preferred_element_type=F32)

    dmas = [dma0, dma1]
    for r_slot in range(N_DEV - 1):
        buf = (r_slot + 1) % 2
        dmas[r_slot + 1].wait()
        if r_slot < N_DEV - 2:
            nxt = wo_dma(r_slot + 2, r_slot % 2)
            nxt.start()
            dmas.append(nxt)
        for h in range(HL):
            rcv = pltpu.make_async_remote_copy(
                src_ref=o_recv.at[r_slot, :, pl.ds(h * DH, DH)],
                dst_ref=o_recv.at[r_slot, :, pl.ds(h * DH, DH)],
                send_sem=send_sems.at[r_slot, h],
                recv_sem=recv_sems.at[r_slot, h],
                device_id=(my,),
                device_id_type=pl.DeviceIdType.MESH,
            )
            rcv.wait_recv()
        orv = o_recv[r_slot]
        for cb in range(2):
            csl = pl.ds(cb * (D // 2), D // 2)
            out_ref[:, csl] += jnp.dot(orv, wo_v[buf][:, cb * (D // 2):
                                                      (cb + 1) * (D // 2)],
                                       preferred_element_type=F32)
    for s in sends:
        s.wait_send()


def kernel(x, Wdkv, Wuk, Wuv, Wq, Wqr, Wkr, Wo):
    x2 = x.reshape(S, D)
    my = lax.axis_index("i")
    wq_blk = lax.dynamic_slice(Wq, (0, my * HB), (D, HB)).astype(BF16)
    wqr_blk = lax.dynamic_slice(
        Wqr, (0, my * HL * DR), (D, HL * DR)).astype(BF16)

    c_full, uk_c, uv_c, q_my, qr_my, kr, wo_b = pl.pallas_call(
        _gather_body,
        out_shape=(
            jax.ShapeDtypeStruct((S, DC_ALL), BF16),
            jax.ShapeDtypeStruct((DC_ALL, HB), BF16),
            jax.ShapeDtypeStruct((DC_ALL, HB), BF16),
            jax.ShapeDtypeStruct((S, HB), BF16),
            jax.ShapeDtypeStruct((S, HL * DR), BF16),
            jax.ShapeDtypeStruct((S, DR), BF16),
            jax.ShapeDtypeStruct((D, D), BF16),
        ),
        in_specs=[pl.BlockSpec(memory_space=pltpu.VMEM)] * 7
        + [pl.BlockSpec(memory_space=pl.ANY)],
        out_specs=(pl.BlockSpec(memory_space=pltpu.VMEM),) * 6
        + (pl.BlockSpec(memory_space=pl.ANY),),
        scratch_shapes=[
            pltpu.VMEM((DC, D), BF16),
            pltpu.VMEM((DC, D), BF16),
            pltpu.VMEM((S, D), BF16),
            pltpu.VMEM((HB, D), F32),
            pltpu.VMEM((HB, D), BF16),
            pltpu.SemaphoreType.DMA((N_DEV - 1, 3)),
            pltpu.SemaphoreType.DMA((N_DEV - 1, 3)),
            pltpu.SemaphoreType.DMA,
            pltpu.SemaphoreType.DMA,
        ],
        compiler_params=pltpu.CompilerParams(collective_id=0),
    )(x2, Wdkv, Wuk, Wuv, wq_blk, wqr_blk, Wkr, Wo)

    out = pl.pallas_call(
        _bc_body,
        out_shape=jax.ShapeDtypeStruct((S, D), F32),
        in_specs=[pl.BlockSpec(memory_space=pltpu.VMEM)] * 6
        + [pl.BlockSpec(memory_space=pl.ANY)],
        out_specs=pl.BlockSpec(memory_space=pltpu.VMEM),
        scratch_shapes=[
            pltpu.VMEM((S, HB), BF16),
            pltpu.VMEM((N_DEV - 1, S, HB), BF16),
            pltpu.VMEM((2, HB, D), BF16),
            pltpu.SemaphoreType.DMA((N_DEV - 1, HL)),
            pltpu.SemaphoreType.DMA((N_DEV - 1, HL)),
            pltpu.SemaphoreType.DMA((2,)),
        ],
        compiler_params=pltpu.CompilerParams(collective_id=1),
    )(c_full, uk_c, uv_c, q_my, qr_my, kr, wo_b)
    return out.reshape(1, S, D)


# device time: 77124 ns/iter; 1.0812x vs baseline; 1.0812x over previous
import jax
import jax.numpy as jnp
from jax import lax
from jax.experimental import pallas as pl
from jax.experimental.pallas import tpu as pltpu

N_DEV = 4
S = 1024
D = 2048
DC = 128
DC_ALL = N_DEV * DC
H = 16
HL = H // N_DEV
DH = 128
HB = HL * DH
DR = 32
SCALE = (DH + DR) ** -0.5
F32 = jnp.float32
BF16 = jnp.bfloat16


def _peer_barrier(peers):
    barrier_sem = pltpu.get_barrier_semaphore()
    for nbr in peers:
        pl.semaphore_signal(barrier_sem, inc=1, device_id=(nbr,),
                            device_id_type=pl.DeviceIdType.MESH)
    pl.semaphore_wait(barrier_sem, len(peers))


def _gather_body(x_ref, wdkv_ref, wuk_ref, wuv_ref, wqb_ref, wqrb_ref,
                 wkr_ref, wo_ref,
                 c_out, uk_out, uv_out, q_out, qr_out, kr_out, wob_out,
                 uk_b, uv_b, xb, wo_s, wo_stage,
                 send_sems, recv_sems, din_sem, dout_sem):
    my = lax.axis_index("i")
    _peer_barrier([(my + j) % N_DEV for j in range(1, N_DEV)])

    uk_b[...] = wuk_ref[...].astype(BF16)
    uv_b[...] = wuv_ref[...].astype(BF16)
    sends = []
    for p_rel in range(1, N_DEV):
        p = (my + p_rel) % N_DEV
        for t, (w_b, dst) in enumerate(((uk_b, uk_out), (uv_b, uv_out))):
            r = pltpu.make_async_remote_copy(
                src_ref=w_b.at[:, pl.ds(p * HB, HB)],
                dst_ref=dst.at[pl.ds(my * DC, DC), :],
                send_sem=send_sems.at[p_rel - 1, t],
                recv_sem=recv_sems.at[3 - p_rel, t],
                device_id=(p,),
                device_id_type=pl.DeviceIdType.MESH,
            )
            r.start()
            sends.append(r)

    xb[...] = x_ref[...].astype(BF16)
    xbv = xb[...]
    c = jnp.dot(xbv, wdkv_ref[...].astype(BF16), preferred_element_type=F32)
    c_out[:, pl.ds(my * DC, DC)] = c.astype(BF16)
    for p_rel in range(1, N_DEV):
        p = (my + p_rel) % N_DEV
        r = pltpu.make_async_remote_copy(
            src_ref=c_out.at[:, pl.ds(my * DC, DC)],
            dst_ref=c_out.at[:, pl.ds(my * DC, DC)],
            send_sem=send_sems.at[p_rel - 1, 2],
            recv_sem=recv_sems.at[3 - p_rel, 2],
            device_id=(p,),
            device_id_type=pl.DeviceIdType.MESH,
        )
        r.start()
        sends.append(r)

    uk_out[pl.ds(my * DC, DC), :] = uk_b[:, pl.ds(my * HB, HB)]
    uv_out[pl.ds(my * DC, DC), :] = uv_b[:, pl.ds(my * HB, HB)]
    kr_out[...] = jnp.dot(xbv, wkr_ref[...].astype(BF16),
                          preferred_element_type=F32).astype(BF16)
    q_out[...] = jnp.dot(xbv, wqb_ref[...],
                         preferred_element_type=F32).astype(BF16)
    qr_out[...] = jnp.dot(xbv, wqrb_ref[...],
                          preferred_element_type=F32).astype(BF16)

    for i in range(N_DEV):
        din = pltpu.make_async_copy(
            wo_ref.at[pl.ds(i * HB, HB), :], wo_s, din_sem)
        din.start()
        din.wait()
        wo_stage[...] = wo_s[...].astype(BF16)
        dout = pltpu.make_async_copy(
            wo_stage, wob_out.at[pl.ds(i * HB, HB), :], dout_sem)
        dout.start()
        dout.wait()

    for r_slot in range(N_DEV - 1):
        o = (my + r_slot + 1) % N_DEV
        for t, dst in enumerate((uk_out, uv_out)):
            rcv = pltpu.make_async_remote_copy(
                src_ref=dst.at[pl.ds(o * DC, DC), :],
                dst_ref=dst.at[pl.ds(o * DC, DC), :],
                send_sem=send_sems.at[r_slot, t],
                recv_sem=recv_sems.at[r_slot, t],
                device_id=(my,),
                device_id_type=pl.DeviceIdType.MESH,
            )
            rcv.wait_recv()
        rcv = pltpu.make_async_remote_copy(
            src_ref=c_out.at[:, pl.ds(o * DC, DC)],
            dst_ref=c_out.at[:, pl.ds(o * DC, DC)],
            send_sem=send_sems.at[r_slot, 2],
            recv_sem=recv_sems.at[r_slot, 2],
            device_id=(my,),
            device_id_type=pl.DeviceIdType.MESH,
        )
        rcv.wait_recv()
    for s in sends:
        s.wait_send()


def _bc_body(c_ref, uk_ref, uv_ref, q_ref, qr_ref, kr_ref, wob_ref,
             out_ref, o_loc, o_recv, wo_v,
             send_sems, recv_sems, dma_sems):
    my = lax.axis_index("i")

    def wo_dma(i, buf):
        idx = (my + i) % N_DEV
        return pltpu.make_async_copy(
            wob_ref.at[pl.ds(idx * HB, HB), :],
            wo_v.at[buf],
            dma_sems.at[buf],
        )

    dma0 = wo_dma(0, 0)
    dma0.start()
    _peer_barrier([(my + j) % N_DEV for j in range(1, N_DEV)])

    cv = c_ref[...]
    krv = kr_ref[...]
    sends = []
    for h in range(HL):
        k_h = jnp.dot(cv, uk_ref[:, h * DH:(h + 1) * DH],
                      preferred_element_type=F32).astype(BF16)
        v_h = jnp.dot(cv, uv_ref[:, h * DH:(h + 1) * DH],
                      preferred_element_type=F32).astype(BF16)
        q_h = q_ref[:, h * DH:(h + 1) * DH]
        qr_h = qr_ref[:, h * DR:(h + 1) * DR]
        s = lax.dot_general(q_h, k_h, (((1,), (1,)), ((), ())),
                            preferred_element_type=F32)
        s += lax.dot_general(qr_h, krv, (((1,), (1,)), ((), ())),
                             preferred_element_type=F32)
        s *= SCALE
        m = jnp.max(s, axis=1, keepdims=True)
        p = jnp.exp(s - m)
        p = (p / jnp.sum(p, axis=1, keepdims=True)).astype(BF16)
        o_loc[:, h * DH:(h + 1) * DH] = jnp.dot(
            p, v_h, preferred_element_type=F32).astype(BF16)
        for p_rel in range(1, N_DEV):
            pd = (my + p_rel) % N_DEV
            r = pltpu.make_async_remote_copy(
                src_ref=o_loc.at[:, pl.ds(h * DH, DH)],
                dst_ref=o_recv.at[3 - p_rel, :, pl.ds(h * DH, DH)],
                send_sem=send_sems.at[p_rel - 1, h],
                recv_sem=recv_sems.at[3 - p_rel, h],
                device_id=(pd,),
                device_id_type=pl.DeviceIdType.MESH,
            )
            r.start()
            sends.append(r)

    dma0.wait()
    dma1 = wo_dma(1, 1)
    dma1.start()
    olv = o_loc[...]
    for cb in range(2):
        csl = pl.ds(cb * (D // 2), D // 2)
        out_ref[:, csl] = jnp.dot(olv, wo_v[0][:, cb * (D // 2):
                                               (cb + 1) * (D // 2)],
                                  preferred_element_type=F32)

    dmas = [dma0, dma1]
    for r_slot in range(N_DEV - 1):
        buf = (r_slot + 1) % 2
        dmas[r_slot + 1].wait()
        if r_slot < N_DEV - 2:
            nxt = wo_dma(r_slot + 2, r_slot % 2)
            nxt.start()
            dmas.append(nxt)
        for h in range(HL):
            rcv = pltpu.make_async_remote_copy(
                src_ref=o_recv.at[r_slot, :, pl.ds(h * DH, DH)],
                dst_ref=o_recv.at[r_slot, :, pl.ds(h * DH, DH)],
                send_sem=send_sems.at[r_slot, h],
                recv_sem=recv_sems.at[r_slot, h],
                device_id=(my,),
                device_id_type=pl.DeviceIdType.MESH,
            )
            rcv.wait_recv()
        orv = o_recv[r_slot]
        for cb in range(2):
            csl = pl.ds(cb * (D // 2), D // 2)
            out_ref[:, csl] += jnp.dot(orv, wo_v[buf][:, cb * (D // 2):
                                                      (cb + 1) * (D // 2)],
                                       preferred_element_type=F32)
    for s in sends:
        s.wait_send()


def kernel(x, Wdkv, Wuk, Wuv, Wq, Wqr, Wkr, Wo):
    x2 = x.reshape(S, D)
    my = lax.axis_index("i")
    wq_blk = lax.dynamic_slice(Wq, (0, my * HB), (D, HB)).astype(BF16)
    wqr_blk = lax.dynamic_slice(
        Wqr, (0, my * HL * DR), (D, HL * DR)).astype(BF16)

    c_full, uk_c, uv_c, q_my, qr_my, kr, wo_b = pl.pallas_call(
        _gather_body,
        out_shape=(
            jax.ShapeDtypeStruct((S, DC_ALL), BF16),
            jax.ShapeDtypeStruct((DC_ALL, HB), BF16),
            jax.ShapeDtypeStruct((DC_ALL, HB), BF16),
            jax.ShapeDtypeStruct((S, HB), BF16),
            jax.ShapeDtypeStruct((S, HL * DR), BF16),
            jax.ShapeDtypeStruct((S, DR), BF16),
            jax.ShapeDtypeStruct((D, D), BF16),
        ),
        in_specs=[pl.BlockSpec(memory_space=pltpu.VMEM)] * 7
        + [pl.BlockSpec(memory_space=pl.ANY)],
        out_specs=(pl.BlockSpec(memory_space=pltpu.VMEM),) * 6
        + (pl.BlockSpec(memory_space=pl.ANY),),
        scratch_shapes=[
            pltpu.VMEM((DC, D), BF16),
            pltpu.VMEM((DC, D), BF16),
            pltpu.VMEM((S, D), BF16),
            pltpu.VMEM((HB, D), F32),
            pltpu.VMEM((HB, D), BF16),
            pltpu.SemaphoreType.DMA((N_DEV - 1, 3)),
            pltpu.SemaphoreType.DMA((N_DEV - 1, 3)),
            pltpu.SemaphoreType.DMA,
            pltpu.SemaphoreType.DMA,
        ],
        compiler_params=pltpu.CompilerParams(collective_id=0),
    )(x2, Wdkv, Wuk, Wuv, wq_blk, wqr_blk, Wkr, Wo)

    out = pl.pallas_call(
        _bc_body,
        out_shape=jax.ShapeDtypeStruct((S, D), F32),
        in_specs=[pl.BlockSpec(memory_space=pltpu.VMEM)] * 6
        + [pl.BlockSpec(memory_space=pl.ANY)],
        out_specs=pl.BlockSpec(memory_space=pltpu.VMEM),
        scratch_shapes=[
            pltpu.VMEM((S, HB), BF16),
            pltpu.VMEM((N_DEV - 1, S, HB), BF16),
            pltpu.VMEM((2, HB, D), BF16),
            pltpu.SemaphoreType.DMA((N_DEV - 1, HL)),
            pltpu.SemaphoreType.DMA((N_DEV - 1, HL)),
            pltpu.SemaphoreType.DMA((2,)),
        ],
        compiler_params=pltpu.CompilerParams(collective_id=1),
    )(c_full, uk_c, uv_c, q_my, qr_my, kr, wo_b)
    return out.reshape(1, S, D)
